# baseline (speedup 1.0000x reference)
import functools

import jax
import jax.numpy as jnp
from jax import lax
from jax.experimental import pallas as pl
from jax.experimental.pallas import tpu as pltpu
from jax.experimental.pallas import tpu_sc as plsc

NUM_FIELDS = 100000
D = 64
N = 409600

NC = 2
NS = 16
NW = NC * NS
B_PER_W = N // NW
C = 128
NCHUNKS = B_PER_W // C
U = 6
NITER = NCHUNKS // U
TAIL = NCHUNKS - NITER * U


def _emb_body(ids_hbm, vals_hbm, tab_hbm, out_hbm,
              idx_all, vals_all, tb0, tb1, tb2, ob0, ob1,
              s_in0, s_in1, s_in2, s_out0, s_out1):
    wid = lax.axis_index("s") * NC + lax.axis_index("c")
    base = wid * B_PER_W

    pltpu.sync_copy(ids_hbm.at[pl.ds(base, B_PER_W)], idx_all)
    pltpu.sync_copy(vals_hbm.at[pl.ds(base, B_PER_W)], vals_all)

    tb_b = (tb0, tb1, tb2)
    ob_b = (ob0, ob1)
    s_in = (s_in0, s_in1, s_in2)
    s_out = (s_out0, s_out1)

    def gather_desc(c, ts):
        off = pl.multiple_of(c * C, C)
        idx_ref = idx_all.at[pl.ds(off, C)]
        return pltpu.make_async_copy(
            tab_hbm.at[idx_ref], tb_b[ts], s_in[ts])

    def store_desc(c, os):
        off = pl.multiple_of((base + c * C) * D, C * D)
        return pltpu.make_async_copy(
            ob_b[os], out_hbm.at[pl.ds(off, C * D)], s_out[os])

    def compute(c, ts, os):
        tb_r, ob_r = tb_b[ts], ob_b[os]
        coff = c * C

        def group(g, _):
            vvec = vals_all[pl.ds(coff + g * 16, 16)]
            for rr in range(16):
                v = vvec[rr]
                r = g * 16 + rr
                for dblk in range(D // 16):
                    fe_sl = pl.ds(dblk * 16, 16)
                    vs_sl = pl.ds(D + dblk * 16, 16)
                    ob_r[pl.ds(r * D + dblk * 16, 16)] = (
                        tb_r[r, fe_sl] + v * tb_r[r, vs_sl])
            return 0

        lax.fori_loop(0, C // 16, group, 0)

    def process(c, ts, os, prefetch_c, wait_guard):
        if wait_guard:
            @pl.when(c >= 2)
            def _drain():
                store_desc(c - 2, os).wait()
        else:
            store_desc(c - 2, os).wait()

        if prefetch_c is not None:
            gather_desc(prefetch_c, (ts + 2) % 3).start()

        gather_desc(c, ts).wait()
        compute(c, ts, os)
        store_desc(c, os).start()

    gather_desc(0, 0).start()
    gather_desc(1, 1).start()

    def six(i, _):
        c0 = U * i
        for u in range(U):
            process(c0 + u, u % 3, u % 2, c0 + u + 2, wait_guard=(u < 2))
        return 0

    lax.fori_loop(0, NITER, six, 0)
    for t in range(TAIL):
        c = NITER * U + t
        pc = c + 2 if c + 2 < NCHUNKS else None
        process(c, c % 3, c % 2, pc, wait_guard=False)
    store_desc(NCHUNKS - 2, (NCHUNKS - 2) % 2).wait()
    store_desc(NCHUNKS - 1, (NCHUNKS - 1) % 2).wait()


@jax.jit
def _emb_lookup(field_ids, values, table):
    mesh = plsc.VectorSubcoreMesh(
        core_axis_name="c", subcore_axis_name="s",
        num_cores=NC, num_subcores=NS)
    f = functools.partial(
        pl.kernel,
        out_type=jax.ShapeDtypeStruct((N * D,), jnp.float32),
        mesh=mesh,
        scratch_types=[
            pltpu.VMEM((B_PER_W,), jnp.int32),
            pltpu.VMEM((B_PER_W,), jnp.float32),
            pltpu.VMEM((C, 2 * D), jnp.float32),
            pltpu.VMEM((C, 2 * D), jnp.float32),
            pltpu.VMEM((C, 2 * D), jnp.float32),
            pltpu.VMEM((C * D,), jnp.float32),
            pltpu.VMEM((C * D,), jnp.float32),
            pltpu.SemaphoreType.DMA,
            pltpu.SemaphoreType.DMA,
            pltpu.SemaphoreType.DMA,
            pltpu.SemaphoreType.DMA,
            pltpu.SemaphoreType.DMA,
        ],
    )(_emb_body)
    return f(field_ids, values, table)


def kernel(field_ids, values, field_embedding, value_scale):
    table = jnp.concatenate([field_embedding, value_scale], axis=1)
    out = _emb_lookup(field_ids.astype(jnp.int32), values, table)
    return lax.optimization_barrier(out).reshape(N, D)


# speedup vs baseline: 1.8784x; 1.8784x over previous
import functools

import jax
import jax.numpy as jnp
from jax import lax
from jax.experimental import pallas as pl
from jax.experimental.pallas import tpu as pltpu
from jax.experimental.pallas import tpu_sc as plsc

NUM_FIELDS = 100000
D = 64
N = 409600

NC = 2
NS = 16
NW = NC * NS
B_PER_W = N // NW
C = 160
NCHUNKS = B_PER_W // C
NPAIRS = NCHUNKS // 2
GL = (128, 32)


def _emb_body(ids_hbm, vals_hbm, tab_hbm, out_hbm,
              idx_all, vals_all, tb0, tb1, ob0, ob1,
              s_in0, s_in1, s_out0, s_out1):
    wid = lax.axis_index("s") * NC + lax.axis_index("c")
    base = wid * B_PER_W

    pltpu.sync_copy(ids_hbm.at[pl.ds(base, B_PER_W)], idx_all)
    pltpu.sync_copy(vals_hbm.at[pl.ds(base, B_PER_W)], vals_all)

    tb_b = (tb0, tb1)
    ob_b = (ob0, ob1)
    s_in = (s_in0, s_in1)
    s_out = (s_out0, s_out1)

    def gather_descs(c, slot):
        descs = []
        j = 0
        for glen in GL:
            off = pl.multiple_of(c * C + j, 8)
            idx_ref = idx_all.at[pl.ds(off, glen)]
            dst = pl.ds(j, glen)
            descs.append(pltpu.make_async_copy(
                tab_hbm.at[idx_ref], tb_b[slot].at[dst], s_in[slot]))
            j += glen
        return descs

    def store_desc(c, slot):
        off = pl.multiple_of(base + c * C, 8)
        return pltpu.make_async_copy(
            ob_b[slot], out_hbm.at[pl.ds(off, C)], s_out[slot])

    def compute(c, slot):
        tb_r, ob_r = tb_b[slot], ob_b[slot]
        coff = c * C

        def group(g, _):
            vvec = vals_all[pl.ds(coff + g * 16, 16)]
            for rr in range(16):
                v = vvec[rr]
                r = g * 16 + rr
                for dblk in range(D // 16):
                    fe_sl = pl.ds(dblk * 16, 16)
                    vs_sl = pl.ds(D + dblk * 16, 16)
                    ob_r[r, fe_sl] = tb_r[r, fe_sl] + v * tb_r[r, vs_sl]
            return 0

        lax.fori_loop(0, C // 16, group, 0)

    for dsc in gather_descs(0, 0):
        dsc.start()

    def pair(i, _):
        for b in (0, 1):
            c = 2 * i + b

            @pl.when(c + 1 < NCHUNKS)
            def _prefetch():
                for dsc in gather_descs(c + 1, 1 - b):
                    dsc.start()

            for dsc in gather_descs(c, b):
                dsc.wait()

            @pl.when(i >= 1)
            def _drain_store():
                store_desc(c - 2, b).wait()

            compute(c, b)
            store_desc(c, b).start()
        return 0

    lax.fori_loop(0, NPAIRS, pair, 0)
    store_desc(NCHUNKS - 2, 0).wait()
    store_desc(NCHUNKS - 1, 1).wait()


@jax.jit
def _emb_lookup(field_ids, values, table):
    mesh = plsc.VectorSubcoreMesh(
        core_axis_name="c", subcore_axis_name="s",
        num_cores=NC, num_subcores=NS)
    f = functools.partial(
        pl.kernel,
        out_type=jax.ShapeDtypeStruct((N, D), jnp.float32),
        mesh=mesh,
        scratch_types=[
            pltpu.VMEM((B_PER_W,), jnp.int32),
            pltpu.VMEM((B_PER_W,), jnp.float32),
            pltpu.VMEM((C, 2 * D), jnp.float32),
            pltpu.VMEM((C, 2 * D), jnp.float32),
            pltpu.VMEM((C, D), jnp.float32),
            pltpu.VMEM((C, D), jnp.float32),
            pltpu.SemaphoreType.DMA,
            pltpu.SemaphoreType.DMA,
            pltpu.SemaphoreType.DMA,
            pltpu.SemaphoreType.DMA,
        ],
    )(_emb_body)
    return f(field_ids, values, table)


def kernel(field_ids, values, field_embedding, value_scale):
    table = jnp.concatenate([field_embedding, value_scale], axis=1)
    out = _emb_lookup(field_ids.astype(jnp.int32), values, table)
    return lax.optimization_barrier(out)
